# Initial kernel scaffold; baseline (speedup 1.0000x reference)
import functools

import jax
import jax.numpy as jnp
from jax import lax
from jax.experimental import pallas as pl
from jax.experimental.pallas import tpu as pltpu
from jax.experimental.pallas import tpu_sc as plsc

NC = 2
NS = 16
NW = NC * NS
CH = 128


@functools.lru_cache(maxsize=None)
def _sc_gather(B, K, D, V):
    b_per_w = B // NW
    n_chunks = (b_per_w * K) // CH
    tp_chunks = b_per_w // CH

    mesh = plsc.VectorSubcoreMesh(core_axis_name="c", subcore_axis_name="s")
    row_f32 = jax.ShapeDtypeStruct((B, D), jnp.float32)

    @functools.partial(
        pl.kernel,
        out_type=(row_f32, row_f32, row_f32),
        mesh=mesh,
        scratch_types=[
            pltpu.VMEM((n_chunks, CH), jnp.int32),
            pltpu.VMEM((n_chunks, CH), jnp.int32),
            pltpu.VMEM((b_per_w, D), jnp.float32),
            pltpu.VMEM((CH, D), jnp.float32),
            pltpu.VMEM((CH, D), jnp.float32),
            pltpu.VMEM((tp_chunks, CH), jnp.int32),
            pltpu.VMEM((CH, D), jnp.float32),
            pltpu.SemaphoreType.DMA,
            pltpu.SemaphoreType.DMA,
        ],
    )
    def gather_kernel(tgt_hbm, ctx_hbm, tidx_hbm, pidx_hbm, nidx_hbm,
                      bidx_hbm, zeros_hbm, t_out, p_out, nsum_out,
                      nidx_v, bidx_v, acc_v, rows_a, rows_b,
                      tpidx_v, tprows_v, sem_a, sem_b):
        wid = lax.axis_index("s") * NC + lax.axis_index("c")
        base_b = wid * b_per_w

        for idx_hbm, table, out in ((tidx_hbm, tgt_hbm, t_out),
                                    (pidx_hbm, ctx_hbm, p_out)):
            pltpu.sync_copy(idx_hbm.at[pl.ds(wid * tp_chunks, tp_chunks)],
                            tpidx_v)
            for c in range(tp_chunks):
                pltpu.sync_copy(table.at[tpidx_v.at[c]], tprows_v)
                pltpu.sync_copy(
                    tprows_v, out.at[pl.ds(base_b + c * CH, CH)])

        pltpu.sync_copy(nidx_hbm.at[pl.ds(wid * n_chunks, n_chunks)], nidx_v)
        pltpu.sync_copy(bidx_hbm, bidx_v)
        pltpu.sync_copy(zeros_hbm, acc_v)

        bufs = (rows_a, rows_b)
        sems = (sem_a, sem_b)
        pending = None
        for c in range(n_chunks):
            cur = pltpu.async_copy(
                ctx_hbm.at[nidx_v.at[c]], bufs[c % 2], sems[c % 2])
            if pending is not None:
                pc, pd = pending
                pd.wait()
                pltpu.sync_copy(bufs[pc % 2], acc_v.at[bidx_v.at[pc]],
                                add=True)
            pending = (c, cur)
        pc, pd = pending
        pd.wait()
        pltpu.sync_copy(bufs[pc % 2], acc_v.at[bidx_v.at[pc]], add=True)

        pltpu.sync_copy(acc_v, nsum_out.at[pl.ds(base_b, b_per_w)])

    return gather_kernel


def _loss_body(t_ref, p_ref, ns_ref, o_ref):
    t = t_ref[...]
    pos = jnp.sum(t * p_ref[...], axis=1)
    neg = jnp.sum(t * ns_ref[...], axis=1)

    def log_sigmoid(x):
        return jnp.minimum(x, 0.0) - jnp.log1p(jnp.exp(-jnp.abs(x)))

    tot = jnp.sum(log_sigmoid(pos) + log_sigmoid(-neg))
    o_ref[0, 0] = -tot / t_ref.shape[0]


def kernel(target_embeddings, context_embeddings, target_block,
           positive_context_block, negative_context_blocks):
    V, D = target_embeddings.shape
    B = target_block.shape[0]
    K = negative_context_blocks.shape[1]
    b_per_w = B // NW

    tidx = target_block.astype(jnp.int32).reshape(-1, CH)
    pidx = positive_context_block.astype(jnp.int32).reshape(-1, CH)
    nidx = negative_context_blocks.astype(jnp.int32).reshape(-1, CH)
    bidx = jnp.repeat(jnp.arange(b_per_w, dtype=jnp.int32), K).reshape(-1, CH)
    zeros = jnp.zeros((b_per_w, D), jnp.float32)

    t, p, nsum = _sc_gather(B, K, D, V)(
        target_embeddings, context_embeddings, tidx, pidx, nidx, bidx, zeros)

    loss = pl.pallas_call(
        _loss_body,
        out_shape=jax.ShapeDtypeStruct((1, 1), jnp.float32),
    )(t, p, nsum)
    return loss[0, 0]


# speedup vs baseline: 5.1583x; 5.1583x over previous
import functools

import jax
import jax.numpy as jnp
from jax import lax
from jax.experimental import pallas as pl
from jax.experimental.pallas import tpu as pltpu
from jax.experimental.pallas import tpu_sc as plsc

NC = 2
NS = 16
NW = NC * NS
CH = 128


@functools.lru_cache(maxsize=None)
def _sc_gather(B, K, D, V):
    b_per_w = B // NW
    n_chunks = (b_per_w * K) // CH
    tp_chunks = b_per_w // CH

    mesh = plsc.VectorSubcoreMesh(core_axis_name="c", subcore_axis_name="s")
    row_f32 = jax.ShapeDtypeStruct((B, D), jnp.float32)

    @functools.partial(
        pl.kernel,
        out_type=(row_f32, row_f32, row_f32),
        mesh=mesh,
        scratch_types=[
            pltpu.VMEM((n_chunks, CH), jnp.int32),
            pltpu.VMEM((n_chunks, CH), jnp.int32),
            pltpu.VMEM_SHARED((NS * b_per_w, D), jnp.float32),
            pltpu.VMEM((CH, D), jnp.float32),
            pltpu.VMEM((CH, D), jnp.float32),
            pltpu.VMEM((tp_chunks, CH), jnp.int32),
            pltpu.VMEM((CH, D), jnp.float32),
            pltpu.SemaphoreType.DMA,
            pltpu.SemaphoreType.DMA,
        ],
        compiler_params=pltpu.CompilerParams(use_tc_tiling_on_sc=False),
    )
    def gather_kernel(tgt_hbm, ctx_hbm, tidx_hbm, pidx_hbm, nidx_hbm,
                      bidx_hbm, zeros_hbm, t_out, p_out, nsum_out,
                      nidx_v, bidx_v, acc_v, rows_a, rows_b,
                      tpidx_v, tprows_v, sem_a, sem_b):
        sid = lax.axis_index("s")
        wid = sid * NC + lax.axis_index("c")
        base_b = wid * b_per_w
        base_sh = sid * b_per_w

        for idx_hbm, table, out in ((tidx_hbm, tgt_hbm, t_out),
                                    (pidx_hbm, ctx_hbm, p_out)):
            pltpu.sync_copy(idx_hbm.at[pl.ds(wid * tp_chunks, tp_chunks)],
                            tpidx_v)
            for c in range(tp_chunks):
                pltpu.sync_copy(table.at[tpidx_v.at[c]], tprows_v)
                pltpu.sync_copy(
                    tprows_v, out.at[pl.ds(base_b + c * CH, CH)])

        pltpu.sync_copy(nidx_hbm.at[pl.ds(wid * n_chunks, n_chunks)], nidx_v)
        pltpu.sync_copy(bidx_hbm.at[pl.ds(sid * n_chunks, n_chunks)], bidx_v)
        pltpu.sync_copy(zeros_hbm, acc_v.at[pl.ds(base_sh, b_per_w)])

        bufs = (rows_a, rows_b)
        sems = (sem_a, sem_b)
        pending = None
        for c in range(n_chunks):
            cur = pltpu.async_copy(
                ctx_hbm.at[nidx_v.at[c]], bufs[c % 2], sems[c % 2])
            if pending is not None:
                pc, pd = pending
                pd.wait()
                pltpu.sync_copy(bufs[pc % 2], acc_v.at[bidx_v.at[pc]],
                                add=True)
            pending = (c, cur)
        pc, pd = pending
        pd.wait()
        pltpu.sync_copy(bufs[pc % 2], acc_v.at[bidx_v.at[pc]], add=True)

        pltpu.sync_copy(acc_v.at[pl.ds(base_sh, b_per_w)],
                        nsum_out.at[pl.ds(base_b, b_per_w)])

    return gather_kernel


def _loss_body(t_ref, p_ref, ns_ref, o_ref):
    t = t_ref[...]
    pos = jnp.sum(t * p_ref[...], axis=1)
    neg = jnp.sum(t * ns_ref[...], axis=1)

    def log_sigmoid(x):
        return jnp.minimum(x, 0.0) - jnp.log1p(jnp.exp(-jnp.abs(x)))

    tot = jnp.sum(log_sigmoid(pos) + log_sigmoid(-neg))
    o_ref[0, 0] = -tot / t_ref.shape[0]


def kernel(target_embeddings, context_embeddings, target_block,
           positive_context_block, negative_context_blocks):
    V, D = target_embeddings.shape
    B = target_block.shape[0]
    K = negative_context_blocks.shape[1]
    b_per_w = B // NW

    tidx = target_block.astype(jnp.int32).reshape(-1, CH)
    pidx = positive_context_block.astype(jnp.int32).reshape(-1, CH)
    nidx = negative_context_blocks.astype(jnp.int32).reshape(-1, CH)
    local = jnp.repeat(jnp.arange(b_per_w, dtype=jnp.int32), K)
    bidx = (jnp.arange(NS, dtype=jnp.int32)[:, None] * b_per_w
            + local[None, :]).reshape(-1, CH)
    zeros = jnp.zeros((b_per_w, D), jnp.float32)

    t, p, nsum = _sc_gather(B, K, D, V)(
        target_embeddings, context_embeddings, tidx, pidx, nidx, bidx, zeros)

    loss = pl.pallas_call(
        _loss_body,
        out_shape=jax.ShapeDtypeStruct((1, 1), jnp.float32),
        out_specs=pl.BlockSpec(memory_space=pltpu.SMEM),
    )(t, p, nsum)
    return loss[0, 0]


# speedup vs baseline: 5.2391x; 1.0157x over previous
import functools

import jax
import jax.numpy as jnp
from jax import lax
from jax.experimental import pallas as pl
from jax.experimental.pallas import tpu as pltpu
from jax.experimental.pallas import tpu_sc as plsc

NC = 2
NS = 16
NW = NC * NS
CH = 128


@functools.lru_cache(maxsize=None)
def _sc_gather(B, K, D, V):
    b_per_w = B // NW
    n_chunks = (b_per_w * K) // CH
    tp_chunks = b_per_w // CH

    mesh = plsc.VectorSubcoreMesh(core_axis_name="c", subcore_axis_name="s")
    row_f32 = jax.ShapeDtypeStruct((B, D), jnp.float32)
    NBUF = 6
    LAG = 3

    @functools.partial(
        pl.kernel,
        out_type=(row_f32, row_f32, row_f32),
        mesh=mesh,
        scratch_types=[
            pltpu.VMEM((n_chunks, CH), jnp.int32),
            pltpu.VMEM((n_chunks, CH), jnp.int32),
            pltpu.VMEM_SHARED((NS * b_per_w, D), jnp.float32),
            [pltpu.VMEM((CH, D), jnp.float32) for _ in range(NBUF)],
            pltpu.VMEM((tp_chunks, CH), jnp.int32),
            pltpu.VMEM((tp_chunks, CH), jnp.int32),
            [pltpu.SemaphoreType.DMA for _ in range(NBUF)],
            [pltpu.SemaphoreType.DMA for _ in range(NBUF)],
        ],
        compiler_params=pltpu.CompilerParams(use_tc_tiling_on_sc=False),
    )
    def gather_kernel(tgt_hbm, ctx_hbm, tidx_hbm, pidx_hbm, nidx_hbm,
                      bidx_hbm, zeros_hbm, t_out, p_out, nsum_out,
                      nidx_v, bidx_v, acc_v, bufs, tidx_v, pidx_v,
                      gsems, osems):
        sid = lax.axis_index("s")
        wid = sid * NC + lax.axis_index("c")
        base_b = wid * b_per_w
        base_sh = sid * b_per_w

        pltpu.sync_copy(tidx_hbm.at[pl.ds(wid * tp_chunks, tp_chunks)],
                        tidx_v)
        pltpu.sync_copy(pidx_hbm.at[pl.ds(wid * tp_chunks, tp_chunks)],
                        pidx_v)
        pltpu.sync_copy(nidx_hbm.at[pl.ds(wid * n_chunks, n_chunks)], nidx_v)
        pltpu.sync_copy(bidx_hbm.at[pl.ds(sid * n_chunks, n_chunks)], bidx_v)
        pltpu.sync_copy(zeros_hbm, acc_v.at[pl.ds(base_sh, b_per_w)])

        work = ([("t", c) for c in range(tp_chunks)]
                + [("p", c) for c in range(tp_chunks)]
                + [("n", c) for c in range(n_chunks)])
        n_items = len(work)

        def issue_gather(kind, c, b):
            if kind == "t":
                return pltpu.async_copy(
                    tgt_hbm.at[tidx_v.at[c]], bufs[b], gsems[b])
            if kind == "p":
                return pltpu.async_copy(
                    ctx_hbm.at[pidx_v.at[c]], bufs[b], gsems[b])
            return pltpu.async_copy(
                ctx_hbm.at[nidx_v.at[c]], bufs[b], gsems[b])

        def issue_out(kind, c, b):
            if kind == "t":
                return pltpu.async_copy(
                    bufs[b], t_out.at[pl.ds(base_b + c * CH, CH)], osems[b])
            if kind == "p":
                return pltpu.async_copy(
                    bufs[b], p_out.at[pl.ds(base_b + c * CH, CH)], osems[b])
            return pltpu.async_copy(
                bufs[b], acc_v.at[bidx_v.at[c]], osems[b], add=True)

        gdescs = [None] * NBUF
        odescs = [None] * NBUF
        for step in range(n_items + LAG):
            if step < n_items:
                b = step % NBUF
                if step >= NBUF:
                    odescs[b].wait()
                kind, c = work[step]
                gdescs[b] = issue_gather(kind, c, b)
            d = step - LAG
            if 0 <= d < n_items:
                b = d % NBUF
                gdescs[b].wait()
                kind, c = work[d]
                odescs[b] = issue_out(kind, c, b)
        for d in range(max(0, n_items - NBUF), n_items):
            odescs[d % NBUF].wait()

        pltpu.sync_copy(acc_v.at[pl.ds(base_sh, b_per_w)],
                        nsum_out.at[pl.ds(base_b, b_per_w)])

    return gather_kernel


def _loss_body(t_ref, p_ref, ns_ref, o_ref):
    t = t_ref[...]
    pos = jnp.sum(t * p_ref[...], axis=1)
    neg = jnp.sum(t * ns_ref[...], axis=1)

    def log_sigmoid(x):
        return jnp.minimum(x, 0.0) - jnp.log1p(jnp.exp(-jnp.abs(x)))

    tot = jnp.sum(log_sigmoid(pos) + log_sigmoid(-neg))
    o_ref[0, 0] = -tot / t_ref.shape[0]


def kernel(target_embeddings, context_embeddings, target_block,
           positive_context_block, negative_context_blocks):
    V, D = target_embeddings.shape
    B = target_block.shape[0]
    K = negative_context_blocks.shape[1]
    b_per_w = B // NW

    tidx = target_block.astype(jnp.int32).reshape(-1, CH)
    pidx = positive_context_block.astype(jnp.int32).reshape(-1, CH)
    nidx = negative_context_blocks.astype(jnp.int32).reshape(-1, CH)
    local = jnp.repeat(jnp.arange(b_per_w, dtype=jnp.int32), K)
    bidx = (jnp.arange(NS, dtype=jnp.int32)[:, None] * b_per_w
            + local[None, :]).reshape(-1, CH)
    zeros = jnp.zeros((b_per_w, D), jnp.float32)

    t, p, nsum = _sc_gather(B, K, D, V)(
        target_embeddings, context_embeddings, tidx, pidx, nidx, bidx, zeros)

    loss = pl.pallas_call(
        _loss_body,
        out_shape=jax.ShapeDtypeStruct((1, 1), jnp.float32),
        out_specs=pl.BlockSpec(memory_space=pltpu.SMEM),
    )(t, p, nsum)
    return loss[0, 0]
